# baseline (speedup 1.0000x reference)
import functools

import jax
import jax.numpy as jnp
from jax import lax
from jax.experimental import pallas as pl
from jax.experimental.pallas import tpu as pltpu
from jax.experimental.pallas import tpu_sc as plsc

VOCAB = 100000
MAXLEN = 2048
EMBED = 64
BATCH = 64
SEQ = 2048

NUM_CORES = 2
NUM_SUBCORES = 16
NW = NUM_CORES * NUM_SUBCORES
EPW = EMBED // NW
LANES = 16
NSTEP = SEQ // LANES
PAIRS = BATCH // 2


def _make_kernel():
    mesh = plsc.VectorSubcoreMesh(core_axis_name="c", subcore_axis_name="s")

    @functools.partial(
        pl.kernel,
        mesh=mesh,
        out_type=jax.ShapeDtypeStruct((BATCH, EMBED, SEQ), jnp.float32),
        compiler_params=pltpu.CompilerParams(
            use_tc_tiling_on_sc=True, needs_layout_passes=False),
        scratch_types=[
            pltpu.VMEM((VOCAB,), jnp.float32),
            pltpu.VMEM((SEQ,), jnp.int32),
            pltpu.VMEM((SEQ,), jnp.int32),
            pltpu.VMEM((SEQ,), jnp.int32),
            pltpu.VMEM((SEQ,), jnp.int32),
            pltpu.VMEM((SEQ,), jnp.float32),
            pltpu.VMEM((SEQ,), jnp.float32),
            pltpu.VMEM((SEQ,), jnp.float32),
            pltpu.VMEM((SEQ,), jnp.float32),
            pltpu.VMEM((SEQ,), jnp.float32),
        ]
        + [pltpu.SemaphoreType.DMA] * 10,
    )
    def emb(x_hbm, tokT_hbm, posT_hbm, outT_hbm, row_v, xv0, xv1, xv2, xv3,
            ov0, ov1, ov2, ov3, pos_r, *sems):
        xbufs = (xv0, xv1, xv2, xv3)
        obufs = (ov0, ov1, ov2, ov3)
        xsem = sems[0:4]
        osem = sems[4:8]
        rsem = sems[8]
        psem = sems[9]
        c = lax.axis_index("c")
        s = lax.axis_index("s")
        wid = s * NUM_CORES + c

        def per_component(t, carry):
            e = wid * EPW + t
            row_cp = pltpu.async_copy(tokT_hbm.at[e], row_v, rsem)
            pos_cp = pltpu.async_copy(posT_hbm.at[e], pos_r, psem)

            def fetch_x(b, slot):
                return pltpu.async_copy(
                    x_hbm.at[pl.ds(b * SEQ, SEQ)], xbufs[slot], xsem[slot])

            def fetch_pair(p):
                s0 = (p % 2) * 2
                return (fetch_x(2 * p, s0), fetch_x(2 * p + 1, s0 + 1))

            xfetch = {0: fetch_pair(0), 1: fetch_pair(1)}
            row_cp.wait()
            pos_cp.wait()

            stores = {}
            for p in range(PAIRS):
                s0 = (p % 2) * 2
                xfetch[p][0].wait()
                xfetch[p][1].wait()
                if p - 2 >= 0:
                    stores[p - 2][0].wait()
                    stores[p - 2][1].wait()
                x0r, x1r = xbufs[s0], xbufs[s0 + 1]
                o0r, o1r = obufs[s0], obufs[s0 + 1]

                @plsc.parallel_loop(0, NSTEP, unroll=8)
                def sbody(i):
                    sl = pl.ds(i * LANES, LANES)
                    pv = pos_r[sl]
                    g0 = plsc.load_gather(row_v, [x0r[sl]])
                    g1 = plsc.load_gather(row_v, [x1r[sl]])
                    o0r[sl] = g0 + pv
                    o1r[sl] = g1 + pv

                if p + 2 < PAIRS:
                    xfetch[p + 2] = fetch_pair(p + 2)
                stores[p] = (
                    pltpu.async_copy(o0r, outT_hbm.at[2 * p, e, :], osem[s0]),
                    pltpu.async_copy(o1r, outT_hbm.at[2 * p + 1, e, :],
                                     osem[s0 + 1]),
                )
            for p in range(PAIRS - 2, PAIRS):
                stores[p][0].wait()
                stores[p][1].wait()
            return carry

        lax.fori_loop(0, EPW, per_component, 0)

    return emb


_emb = _make_kernel()


def kernel(x, token_table, pos_table):
    x_flat = x.reshape(BATCH * SEQ).astype(jnp.int32)
    outT = _emb(x_flat, token_table.T, pos_table.T)
    return outT.transpose(0, 2, 1)


# speedup vs baseline: 1.0001x; 1.0001x over previous
import functools

import jax
import jax.numpy as jnp
from jax import lax
from jax.experimental import pallas as pl
from jax.experimental.pallas import tpu as pltpu
from jax.experimental.pallas import tpu_sc as plsc

VOCAB = 100000
MAXLEN = 2048
EMBED = 64
BATCH = 64
SEQ = 2048

NUM_CORES = 2
NUM_SUBCORES = 16
NW = NUM_CORES * NUM_SUBCORES
EPW = EMBED // NW
LANES = 16
NSTEP = SEQ // LANES
PAIRS = BATCH // 2


def _make_kernel():
    mesh = plsc.VectorSubcoreMesh(core_axis_name="c", subcore_axis_name="s")

    @functools.partial(
        pl.kernel,
        mesh=mesh,
        out_type=jax.ShapeDtypeStruct((BATCH, EMBED, SEQ), jnp.float32),
        compiler_params=pltpu.CompilerParams(
            use_tc_tiling_on_sc=True, needs_layout_passes=False),
        scratch_types=[
            pltpu.VMEM((VOCAB,), jnp.float32),
            pltpu.VMEM((2 * SEQ,), jnp.int32),
            pltpu.VMEM((2 * SEQ,), jnp.int32),
            pltpu.VMEM((2, SEQ), jnp.float32),
            pltpu.VMEM((2, SEQ), jnp.float32),
            pltpu.VMEM((SEQ,), jnp.float32),
        ]
        + [pltpu.SemaphoreType.DMA] * 6,
    )
    def emb(x_hbm, tokT_hbm, posT_hbm, outT_hbm, row_v, xv0, xv1,
            ov0, ov1, pos_r, *sems):
        xbufs = (xv0, xv1)
        obufs = (ov0, ov1)
        xsem = sems[0:2]
        osem = sems[2:4]
        rsem = sems[4]
        psem = sems[5]
        c = lax.axis_index("c")
        s = lax.axis_index("s")
        wid = s * NUM_CORES + c

        def per_component(t, carry):
            e = wid * EPW + t
            row_cp = pltpu.async_copy(tokT_hbm.at[e], row_v, rsem)
            pos_cp = pltpu.async_copy(posT_hbm.at[e], pos_r, psem)

            def fetch_pair(p):
                return pltpu.async_copy(
                    x_hbm.at[pl.ds(2 * p * SEQ, 2 * SEQ)], xbufs[p % 2],
                    xsem[p % 2])

            xfetch = {0: fetch_pair(0), 1: fetch_pair(1)}
            row_cp.wait()
            pos_cp.wait()

            stores = {}
            for p in range(PAIRS):
                sl0 = p % 2
                xfetch[p].wait()
                if p - 2 >= 0:
                    stores[p - 2].wait()
                xr = xbufs[sl0]
                orow = obufs[sl0]

                @plsc.parallel_loop(0, NSTEP, unroll=4)
                def sbody(i):
                    sl = pl.ds(i * LANES, LANES)
                    pv = pos_r[sl]
                    g0 = plsc.load_gather(row_v, [xr[pl.ds(i * LANES, LANES)]])
                    g1 = plsc.load_gather(
                        row_v, [xr[pl.ds(SEQ + i * LANES, LANES)]])
                    orow[0, sl] = g0 + pv
                    orow[1, sl] = g1 + pv

                if p + 2 < PAIRS:
                    xfetch[p + 2] = fetch_pair(p + 2)
                stores[p] = pltpu.async_copy(
                    orow, outT_hbm.at[pl.ds(2 * p, 2), e, :], osem[sl0])
            for p in range(PAIRS - 2, PAIRS):
                stores[p].wait()
            return carry

        lax.fori_loop(0, EPW, per_component, 0)

    return emb


_emb = _make_kernel()


def kernel(x, token_table, pos_table):
    x_flat = x.reshape(BATCH * SEQ).astype(jnp.int32)
    outT = _emb(x_flat, token_table.T, pos_table.T)
    return outT.transpose(0, 2, 1)


# speedup vs baseline: 1.0308x; 1.0306x over previous
import functools

import jax
import jax.numpy as jnp
from jax import lax
from jax.experimental import pallas as pl
from jax.experimental.pallas import tpu as pltpu
from jax.experimental.pallas import tpu_sc as plsc

VOCAB = 100000
MAXLEN = 2048
EMBED = 64
BATCH = 64
SEQ = 2048

NUM_CORES = 2
NUM_SUBCORES = 16
NW = NUM_CORES * NUM_SUBCORES
EPW = EMBED // NW
LANES = 16
NSTEP = SEQ // LANES
PAIRS = BATCH // 2


def _make_kernel():
    mesh = plsc.VectorSubcoreMesh(core_axis_name="c", subcore_axis_name="s")

    @functools.partial(
        pl.kernel,
        mesh=mesh,
        out_type=jax.ShapeDtypeStruct((BATCH, EMBED, SEQ), jnp.float32),
        compiler_params=pltpu.CompilerParams(
            use_tc_tiling_on_sc=True, needs_layout_passes=False),
        scratch_types=[
            pltpu.VMEM((VOCAB,), jnp.float32),
            pltpu.VMEM((4 * SEQ,), jnp.int32),
            pltpu.VMEM((4 * SEQ,), jnp.int32),
            pltpu.VMEM((SEQ,), jnp.float32),
            pltpu.VMEM((SEQ,), jnp.float32),
            pltpu.VMEM((SEQ,), jnp.float32),
            pltpu.VMEM((SEQ,), jnp.float32),
            pltpu.VMEM((SEQ,), jnp.float32),
        ]
        + [pltpu.SemaphoreType.DMA] * 8,
    )
    def emb(x_hbm, tokT_hbm, posT_hbm, outT_hbm, row_v, xv0, xv1,
            ov0, ov1, ov2, ov3, pos_r, *sems):
        xbufs = (xv0, xv1)
        obufs = (ov0, ov1, ov2, ov3)
        xsem = sems[0:2]
        osem = sems[2:6]
        rsem = sems[6]
        psem = sems[7]
        c = lax.axis_index("c")
        s = lax.axis_index("s")
        wid = s * NUM_CORES + c

        def fetch_quad(t, q):
            return pltpu.async_copy(
                x_hbm.at[pl.ds(4 * q * SEQ, 4 * SEQ)], xbufs[q % 2],
                xsem[q % 2])

        stores = {}
        row_cp = pltpu.async_copy(tokT_hbm.at[wid * EPW], row_v, rsem)
        pos_cp = pltpu.async_copy(posT_hbm.at[wid * EPW], pos_r, psem)
        xfetch = {(0, 0): fetch_quad(0, 0), (0, 1): fetch_quad(0, 1)}
        for t in range(EPW):
            e = wid * EPW + t
            row_cp.wait()
            pos_cp.wait()
            for p in range(PAIRS):
                g = t * PAIRS + p
                q = p // 2
                s0 = (p % 2) * 2
                if p % 2 == 0:
                    xfetch[(t, q)].wait()
                    nxt = (t, q + 1) if q + 1 < PAIRS // 2 else (t + 1, 0)
                    if nxt not in xfetch and nxt[0] < EPW:
                        xfetch[nxt] = fetch_quad(*nxt)
                if g - 2 >= 0:
                    stores[g - 2][0].wait()
                    stores[g - 2][1].wait()
                xr = xbufs[q % 2]
                xo = (p % 2) * 2 * SEQ
                o0r, o1r = obufs[s0], obufs[s0 + 1]

                @plsc.parallel_loop(0, NSTEP, unroll=4)
                def sbody(i):
                    sl = pl.ds(i * LANES, LANES)
                    pv = pos_r[sl]
                    g0 = plsc.load_gather(
                        row_v, [xr[pl.ds(xo + i * LANES, LANES)]])
                    g1 = plsc.load_gather(
                        row_v, [xr[pl.ds(xo + SEQ + i * LANES, LANES)]])
                    o0r[sl] = g0 + pv
                    o1r[sl] = g1 + pv

                if t + 1 < EPW and p == PAIRS - 1:
                    row_cp = pltpu.async_copy(tokT_hbm.at[e + 1], row_v, rsem)
                    pos_cp = pltpu.async_copy(posT_hbm.at[e + 1], pos_r, psem)
                stores[g] = (
                    pltpu.async_copy(o0r, outT_hbm.at[2 * p, e, :], osem[s0]),
                    pltpu.async_copy(o1r, outT_hbm.at[2 * p + 1, e, :],
                                     osem[s0 + 1]),
                )
        for g in range(EPW * PAIRS - 2, EPW * PAIRS):
            stores[g][0].wait()
            stores[g][1].wait()

    return emb


_emb = _make_kernel()


def kernel(x, token_table, pos_table):
    x_flat = x.reshape(BATCH * SEQ).astype(jnp.int32)
    outT = _emb(x_flat, token_table.T, pos_table.T)
    return outT.transpose(0, 2, 1)
